# Initial kernel scaffold; baseline (speedup 1.0000x reference)
import jax
import jax.numpy as jnp
from jax.experimental import pallas as pl


def kernel(pred_mz, pred_intensity, target_mz, target_intensity, target_mask):
    raise NotImplementedError("write your pallas kernel here")


# speedup vs baseline: 20.0521x; 20.0521x over previous
import functools

import jax
import jax.numpy as jnp
from jax import lax
from jax.experimental import pallas as pl
from jax.experimental.pallas import tpu as pltpu
from jax.experimental.pallas import tpu_sc as plsc

B = 4096
P = 200
L = 16
PP = 208
VPR = PP // L
NUM_BINS = 2000
NBP = 2048
NC = 2
NS = 16
NW = NC * NS
RPW = B // NW
RCH = 16
NCH = RPW // RCH
F32 = jnp.float32

_sc_mesh = plsc.VectorSubcoreMesh(core_axis_name="c", subcore_axis_name="s")


@functools.partial(
    pl.kernel,
    mesh=_sc_mesh,
    compiler_params=pltpu.CompilerParams(needs_layout_passes=False),
    out_type=[
        jax.ShapeDtypeStruct((B * L,), F32),
        jax.ShapeDtypeStruct((B * L,), F32),
        jax.ShapeDtypeStruct((B * L,), F32),
    ],
    scratch_types=[
        pltpu.VMEM((RCH * PP,), F32),
        pltpu.VMEM((RCH * PP,), F32),
        pltpu.VMEM((RCH * PP,), F32),
        pltpu.VMEM((RCH * PP,), F32),
        pltpu.VMEM((RCH * PP,), F32),
        pltpu.VMEM((NBP,), F32),
        pltpu.VMEM((NBP,), F32),
        pltpu.VMEM((PP,), jnp.int32),
        pltpu.VMEM((PP,), jnp.int32),
        pltpu.VMEM((RCH * L,), F32),
        pltpu.VMEM((RCH * L,), F32),
        pltpu.VMEM((RCH * L,), F32),
    ],
)
def _sc_hist(pmz_h, pint_h, tmz_h, tint_h, tmask_h,
             dot_h, na_h, nb_h,
             v_pmz, v_pint, v_tmz, v_tint, v_tmask,
             hp, ht, pb_buf, tb_buf, r_dot, r_na, r_nb):
    wid = lax.axis_index("s") * NC + lax.axis_index("c")
    zero16 = jnp.zeros((L,), F32)

    def zero_body(i, carry):
        hp[pl.ds(i * L, L)] = zero16
        ht[pl.ds(i * L, L)] = zero16
        return carry

    lax.fori_loop(0, NBP // L, zero_body, 0)

    def row_body(r, carry):
        roff = r * PP
        for j in range(VPR):
            sl = pl.ds(roff + j * L, L)
            bsl = pl.ds(j * L, L)
            pb = jnp.minimum(
                jnp.maximum((v_pmz[sl] * 2000.0).astype(jnp.int32), 0), NUM_BINS - 1)
            pb_buf[bsl] = pb
            plsc.addupdate_scatter(hp, [pb], v_pint[sl])
            tb = jnp.minimum(
                jnp.maximum((v_tmz[sl] * 2000.0).astype(jnp.int32), 0), NUM_BINS - 1)
            tb_buf[bsl] = tb
            tm = v_tint[sl] * v_tmask[sl]
            v_tint[sl] = tm
            plsc.addupdate_scatter(ht, [tb], tm)
        acc_d = zero16
        acc_a = zero16
        acc_b = zero16
        for j in range(VPR):
            sl = pl.ds(roff + j * L, L)
            bsl = pl.ds(j * L, L)
            pb = pb_buf[bsl]
            tb = tb_buf[bsl]
            acc_a = acc_a + v_pint[sl] * plsc.load_gather(hp, [pb])
            tm = v_tint[sl]
            acc_b = acc_b + tm * plsc.load_gather(ht, [tb])
            acc_d = acc_d + tm * plsc.load_gather(hp, [tb])
        rsl = pl.ds(r * L, L)
        r_dot[rsl] = acc_d
        r_na[rsl] = acc_a
        r_nb[rsl] = acc_b
        for j in range(VPR):
            bsl = pl.ds(j * L, L)
            plsc.store_scatter(hp, [pb_buf[bsl]], zero16)
            plsc.store_scatter(ht, [tb_buf[bsl]], zero16)
        return carry

    def chunk_body(c, carry):
        rowbase = wid * RPW + c * RCH
        ibase = rowbase * PP
        pltpu.sync_copy(pmz_h.at[pl.ds(ibase, RCH * PP)], v_pmz)
        pltpu.sync_copy(pint_h.at[pl.ds(ibase, RCH * PP)], v_pint)
        pltpu.sync_copy(tmz_h.at[pl.ds(ibase, RCH * PP)], v_tmz)
        pltpu.sync_copy(tint_h.at[pl.ds(ibase, RCH * PP)], v_tint)
        pltpu.sync_copy(tmask_h.at[pl.ds(ibase, RCH * PP)], v_tmask)
        lax.fori_loop(0, RCH, row_body, 0)
        obase = rowbase * L
        pltpu.sync_copy(r_dot, dot_h.at[pl.ds(obase, RCH * L)])
        pltpu.sync_copy(r_na, na_h.at[pl.ds(obase, RCH * L)])
        pltpu.sync_copy(r_nb, nb_h.at[pl.ds(obase, RCH * L)])
        return carry

    lax.fori_loop(0, NCH, chunk_body, 0)


def _tc_finish_body(dp_ref, na_ref, nb_ref, o_ref):
    jj = lax.broadcasted_iota(jnp.int32, (128, 8), 0)
    kk = lax.broadcasted_iota(jnp.int32, (128, 8), 1)
    m = (jj // L == kk).astype(F32)
    dot = jnp.dot(dp_ref[...], m, preferred_element_type=F32)
    na2 = jnp.dot(na_ref[...], m, preferred_element_type=F32)
    nb2 = jnp.dot(nb_ref[...], m, preferred_element_type=F32)
    na = jnp.maximum(jnp.sqrt(na2), 1e-8)
    nb = jnp.maximum(jnp.sqrt(nb2), 1e-8)
    cos = jnp.clip(dot / (na * nb), -1.0, 1.0)
    a = jnp.abs(cos)
    p = jnp.float32(-0.0012624911)
    for c in (0.0066700901, -0.0170881256, 0.0308918810, -0.0501743046,
              0.0889789874, -0.2145988016, 1.5707963050):
        p = p * a + jnp.float32(c)
    r = jnp.sqrt(jnp.maximum(1.0 - a, 0.0)) * p
    ang = jnp.where(cos < 0.0, jnp.float32(jnp.pi) - r, r)
    o_ref[0, 0] = jnp.sum(ang) / (B * jnp.pi)


def _tc_finish(dp, na, nb):
    return pl.pallas_call(
        _tc_finish_body,
        out_shape=jax.ShapeDtypeStruct((1, 1), F32),
        out_specs=pl.BlockSpec(memory_space=pltpu.SMEM),
    )(dp.reshape(B * L // 128, 128), na.reshape(B * L // 128, 128),
      nb.reshape(B * L // 128, 128))


def kernel(pred_mz, pred_intensity, target_mz, target_intensity, target_mask):
    def flat(x):
        return jnp.pad(x, ((0, 0), (0, PP - P))).reshape(-1)

    dot_p, na_p, nb_p = _sc_hist(
        flat(pred_mz), flat(pred_intensity), flat(target_mz),
        flat(target_intensity), flat(target_mask))
    return _tc_finish(dot_p, na_p, nb_p)[0, 0]


# speedup vs baseline: 24.9706x; 1.2453x over previous
import functools

import jax
import jax.numpy as jnp
from jax import lax
from jax.experimental import pallas as pl
from jax.experimental.pallas import tpu as pltpu
from jax.experimental.pallas import tpu_sc as plsc

B = 4096
P = 200
L = 16
VPR = (P + L - 1) // L
NUM_BINS = 2000
NBP = 2048
NC = 2
NS = 16
NW = NC * NS
RPW = B // NW
RCH = 32
NCH = RPW // RCH
CW = RCH * P
F32 = jnp.float32

_sc_mesh = plsc.VectorSubcoreMesh(core_axis_name="c", subcore_axis_name="s")


@functools.partial(
    pl.kernel,
    mesh=_sc_mesh,
    compiler_params=pltpu.CompilerParams(needs_layout_passes=False),
    out_type=[
        jax.ShapeDtypeStruct((B * L,), F32),
        jax.ShapeDtypeStruct((B * L,), F32),
        jax.ShapeDtypeStruct((B * L,), F32),
    ],
    scratch_types=[
        pltpu.VMEM((2, CW + 8), F32),
        pltpu.VMEM((2, CW + 8), F32),
        pltpu.VMEM((2, CW + 8), F32),
        pltpu.VMEM((2, CW + 8), F32),
        pltpu.VMEM((2, CW + 8), F32),
        pltpu.VMEM((NBP,), F32),
        pltpu.VMEM((NBP,), F32),
        pltpu.VMEM((VPR * L,), jnp.int32),
        pltpu.VMEM((VPR * L,), jnp.int32),
        pltpu.VMEM((VPR * L,), F32),
        pltpu.VMEM((RPW * L,), F32),
        pltpu.VMEM((RPW * L,), F32),
        pltpu.VMEM((RPW * L,), F32),
        pltpu.SemaphoreType.DMA,
        pltpu.SemaphoreType.DMA,
    ],
)
def _sc_hist(pmz_h, pint_h, tmz_h, tint_h, tmask_h,
             dot_h, na_h, nb_h,
             v_pmz, v_pint, v_tmz, v_tint, v_tmask,
             hp, ht, pb_buf, tb_buf, tm_buf, r_dot, r_na, r_nb,
             sem0, sem1):
    wid = lax.axis_index("s") * NC + lax.axis_index("c")
    zero16 = jnp.zeros((L,), F32)
    m8 = lax.broadcasted_iota(jnp.int32, (L,), 0) < (P - (VPR - 1) * L)

    def zero_body(i, carry):
        hp[pl.ds(i * L, L)] = zero16
        ht[pl.ds(i * L, L)] = zero16
        return carry

    lax.fori_loop(0, NBP // L, zero_body, 0)

    hbm_in = (pmz_h, pint_h, tmz_h, tint_h, tmask_h)
    bufs = (v_pmz, v_pint, v_tmz, v_tint, v_tmask)

    def issue(c, s, sem):
        base = (wid * RPW + c * RCH) * P
        return [pltpu.async_copy(h.at[pl.ds(base, CW)], b.at[s, pl.ds(0, CW)], sem)
                for h, b in zip(hbm_in, bufs)]

    def make_row_body(s):
        def row_body(r, carry):
            roff = r * P
            for j in range(VPR):
                sl = pl.ds(roff + j * L, L)
                bsl = pl.ds(j * L, L)
                pb = jnp.minimum(
                    jnp.maximum((v_pmz[s, sl] * 2000.0).astype(jnp.int32), 0),
                    NUM_BINS - 1)
                pb_buf[bsl] = pb
                pint = v_pint[s, sl]
                tm = v_tint[s, sl] * v_tmask[s, sl]
                if j == VPR - 1:
                    pint = jnp.where(m8, pint, 0.0)
                    tm = jnp.where(m8, tm, 0.0)
                plsc.addupdate_scatter(hp, [pb], pint)
                tb = jnp.minimum(
                    jnp.maximum((v_tmz[s, sl] * 2000.0).astype(jnp.int32), 0),
                    NUM_BINS - 1)
                tb_buf[bsl] = tb
                tm_buf[bsl] = tm
                plsc.addupdate_scatter(ht, [tb], tm)
            acc_d = zero16
            acc_a = zero16
            acc_b = zero16
            for j in range(VPR):
                sl = pl.ds(roff + j * L, L)
                bsl = pl.ds(j * L, L)
                pb = pb_buf[bsl]
                tb = tb_buf[bsl]
                pint = v_pint[s, sl]
                if j == VPR - 1:
                    pint = jnp.where(m8, pint, 0.0)
                acc_a = acc_a + pint * plsc.load_gather(hp, [pb])
                tm = tm_buf[bsl]
                acc_b = acc_b + tm * plsc.load_gather(ht, [tb])
                acc_d = acc_d + tm * plsc.load_gather(hp, [tb])
            rsl = pl.ds(carry + r * L, L)
            r_dot[rsl] = acc_d
            r_na[rsl] = acc_a
            r_nb[rsl] = acc_b
            for j in range(VPR):
                bsl = pl.ds(j * L, L)
                plsc.store_scatter(hp, [pb_buf[bsl]], zero16)
                plsc.store_scatter(ht, [tb_buf[bsl]], zero16)
            return carry

        return row_body

    handles = issue(0, 0, sem0)
    for c in range(NCH):
        s = c % 2
        for hdl in handles:
            hdl.wait()
        if c + 1 < NCH:
            handles = issue(c + 1, 1 - s, sem1 if s == 0 else sem0)
        lax.fori_loop(0, RCH, make_row_body(s), c * RCH * L)
    obase = wid * RPW * L
    pltpu.sync_copy(r_dot, dot_h.at[pl.ds(obase, RPW * L)])
    pltpu.sync_copy(r_na, na_h.at[pl.ds(obase, RPW * L)])
    pltpu.sync_copy(r_nb, nb_h.at[pl.ds(obase, RPW * L)])


def _tc_finish_body(dp_ref, na_ref, nb_ref, o_ref):
    jj = lax.broadcasted_iota(jnp.int32, (128, 8), 0)
    kk = lax.broadcasted_iota(jnp.int32, (128, 8), 1)
    m = (jj // L == kk).astype(F32)
    dot = jnp.dot(dp_ref[...], m, preferred_element_type=F32)
    na2 = jnp.dot(na_ref[...], m, preferred_element_type=F32)
    nb2 = jnp.dot(nb_ref[...], m, preferred_element_type=F32)
    na = jnp.maximum(jnp.sqrt(na2), 1e-8)
    nb = jnp.maximum(jnp.sqrt(nb2), 1e-8)
    cos = jnp.clip(dot / (na * nb), -1.0, 1.0)
    a = jnp.abs(cos)
    p = jnp.float32(-0.0012624911)
    for c in (0.0066700901, -0.0170881256, 0.0308918810, -0.0501743046,
              0.0889789874, -0.2145988016, 1.5707963050):
        p = p * a + jnp.float32(c)
    r = jnp.sqrt(jnp.maximum(1.0 - a, 0.0)) * p
    ang = jnp.where(cos < 0.0, jnp.float32(jnp.pi) - r, r)
    o_ref[0, 0] = jnp.sum(ang) / (B * jnp.pi)


def _tc_finish(dp, na, nb):
    return pl.pallas_call(
        _tc_finish_body,
        out_shape=jax.ShapeDtypeStruct((1, 1), F32),
        out_specs=pl.BlockSpec(memory_space=pltpu.SMEM),
    )(dp.reshape(B * L // 128, 128), na.reshape(B * L // 128, 128),
      nb.reshape(B * L // 128, 128))


def kernel(pred_mz, pred_intensity, target_mz, target_intensity, target_mask):
    dot_p, na_p, nb_p = _sc_hist(
        pred_mz.reshape(-1), pred_intensity.reshape(-1), target_mz.reshape(-1),
        target_intensity.reshape(-1), target_mask.reshape(-1))
    return _tc_finish(dot_p, na_p, nb_p)[0, 0]


# speedup vs baseline: 32.6344x; 1.3069x over previous
import functools

import jax
import jax.numpy as jnp
from jax import lax
from jax.experimental import pallas as pl
from jax.experimental.pallas import tpu as pltpu
from jax.experimental.pallas import tpu_sc as plsc

B = 4096
P = 200
L = 16
VPR = (P + L - 1) // L
NUM_BINS = 2000
NBP = 2048
NC = 2
NS = 16
NW = NC * NS
RPW = B // NW
RCH = 32
NCH = RPW // RCH
F32 = jnp.float32
COLS = tuple(j * L for j in range(VPR - 1)) + (P - L,)

_sc_mesh = plsc.VectorSubcoreMesh(core_axis_name="c", subcore_axis_name="s")


@functools.partial(
    pl.kernel,
    mesh=_sc_mesh,
    compiler_params=pltpu.CompilerParams(needs_layout_passes=False),
    out_type=[
        jax.ShapeDtypeStruct((B * L,), F32),
        jax.ShapeDtypeStruct((B * L,), F32),
        jax.ShapeDtypeStruct((B * L,), F32),
    ],
    scratch_types=[
        pltpu.VMEM((2, RCH, P), F32),
        pltpu.VMEM((2, RCH, P), F32),
        pltpu.VMEM((2, RCH, P), F32),
        pltpu.VMEM((2, RCH, P), F32),
        pltpu.VMEM((2, RCH, P), F32),
        pltpu.VMEM((NBP,), F32),
        pltpu.VMEM((NBP,), F32),
        pltpu.VMEM((VPR * L,), jnp.int32),
        pltpu.VMEM((VPR * L,), jnp.int32),
        pltpu.VMEM((VPR * L,), F32),
        pltpu.VMEM((RPW * L,), F32),
        pltpu.VMEM((RPW * L,), F32),
        pltpu.VMEM((RPW * L,), F32),
        pltpu.SemaphoreType.DMA,
        pltpu.SemaphoreType.DMA,
    ],
)
def _sc_hist(pmz_h, pint_h, tmz_h, tint_h, tmask_h,
             dot_h, na_h, nb_h,
             v_pmz, v_pint, v_tmz, v_tint, v_tmask,
             hp, ht, pb_buf, tb_buf, tm_buf, r_dot, r_na, r_nb,
             sem0, sem1):
    wid = lax.axis_index("s") * NC + lax.axis_index("c")
    zero16 = jnp.zeros((L,), F32)
    m_keep = lax.broadcasted_iota(jnp.int32, (L,), 0) >= (VPR * L - P)

    def zero_body(i, carry):
        hp[pl.ds(i * L, L)] = zero16
        ht[pl.ds(i * L, L)] = zero16
        return carry

    lax.fori_loop(0, NBP // L, zero_body, 0)

    hbm_in = (pmz_h, pint_h, tmz_h, tint_h, tmask_h)
    bufs = (v_pmz, v_pint, v_tmz, v_tint, v_tmask)

    def issue(c, s, sem):
        base = wid * RPW + c * RCH
        return [pltpu.async_copy(h.at[pl.ds(base, RCH)], b.at[s], sem)
                for h, b in zip(hbm_in, bufs)]

    def make_row_body(s):
        def row_body(r, carry):
            for j in range(VPR):
                sl = pl.ds(COLS[j], L)
                bsl = pl.ds(j * L, L)
                pb = jnp.minimum(
                    jnp.maximum((v_pmz[s, r, sl] * 2000.0).astype(jnp.int32), 0),
                    NUM_BINS - 1)
                pb_buf[bsl] = pb
                pint = v_pint[s, r, sl]
                tm = v_tint[s, r, sl] * v_tmask[s, r, sl]
                if j == VPR - 1:
                    pint = jnp.where(m_keep, pint, 0.0)
                    tm = jnp.where(m_keep, tm, 0.0)
                plsc.addupdate_scatter(hp, [pb], pint)
                tb = jnp.minimum(
                    jnp.maximum((v_tmz[s, r, sl] * 2000.0).astype(jnp.int32), 0),
                    NUM_BINS - 1)
                tb_buf[bsl] = tb
                tm_buf[bsl] = tm
                plsc.addupdate_scatter(ht, [tb], tm)
            acc_d = zero16
            acc_a = zero16
            acc_b = zero16
            for j in range(VPR):
                sl = pl.ds(COLS[j], L)
                bsl = pl.ds(j * L, L)
                pb = pb_buf[bsl]
                tb = tb_buf[bsl]
                pint = v_pint[s, r, sl]
                if j == VPR - 1:
                    pint = jnp.where(m_keep, pint, 0.0)
                acc_a = acc_a + pint * plsc.load_gather(hp, [pb])
                tm = tm_buf[bsl]
                acc_b = acc_b + tm * plsc.load_gather(ht, [tb])
                acc_d = acc_d + tm * plsc.load_gather(hp, [tb])
            rsl = pl.ds(carry + r * L, L)
            r_dot[rsl] = acc_d
            r_na[rsl] = acc_a
            r_nb[rsl] = acc_b
            for j in range(VPR):
                bsl = pl.ds(j * L, L)
                plsc.store_scatter(hp, [pb_buf[bsl]], zero16)
                plsc.store_scatter(ht, [tb_buf[bsl]], zero16)
            return carry

        return row_body

    handles = issue(0, 0, sem0)
    for c in range(NCH):
        s = c % 2
        for hdl in handles:
            hdl.wait()
        if c + 1 < NCH:
            handles = issue(c + 1, 1 - s, sem1 if s == 0 else sem0)
        lax.fori_loop(0, RCH, make_row_body(s), c * RCH * L)
    obase = wid * RPW * L
    pltpu.sync_copy(r_dot, dot_h.at[pl.ds(obase, RPW * L)])
    pltpu.sync_copy(r_na, na_h.at[pl.ds(obase, RPW * L)])
    pltpu.sync_copy(r_nb, nb_h.at[pl.ds(obase, RPW * L)])


def _tc_finish_body(dp_ref, na_ref, nb_ref, o_ref):
    jj = lax.broadcasted_iota(jnp.int32, (128, 8), 0)
    kk = lax.broadcasted_iota(jnp.int32, (128, 8), 1)
    m = (jj // L == kk).astype(F32)
    dot = jnp.dot(dp_ref[...], m, preferred_element_type=F32)
    na2 = jnp.dot(na_ref[...], m, preferred_element_type=F32)
    nb2 = jnp.dot(nb_ref[...], m, preferred_element_type=F32)
    na = jnp.maximum(jnp.sqrt(na2), 1e-8)
    nb = jnp.maximum(jnp.sqrt(nb2), 1e-8)
    cos = jnp.clip(dot / (na * nb), -1.0, 1.0)
    a = jnp.abs(cos)
    p = jnp.float32(-0.0012624911)
    for c in (0.0066700901, -0.0170881256, 0.0308918810, -0.0501743046,
              0.0889789874, -0.2145988016, 1.5707963050):
        p = p * a + jnp.float32(c)
    r = jnp.sqrt(jnp.maximum(1.0 - a, 0.0)) * p
    ang = jnp.where(cos < 0.0, jnp.float32(jnp.pi) - r, r)
    o_ref[0, 0] = jnp.sum(ang) / (B * jnp.pi)


def _tc_finish(dp, na, nb):
    return pl.pallas_call(
        _tc_finish_body,
        out_shape=jax.ShapeDtypeStruct((1, 1), F32),
        out_specs=pl.BlockSpec(memory_space=pltpu.SMEM),
    )(dp.reshape(B * L // 128, 128), na.reshape(B * L // 128, 128),
      nb.reshape(B * L // 128, 128))


def kernel(pred_mz, pred_intensity, target_mz, target_intensity, target_mask):
    dot_p, na_p, nb_p = _sc_hist(
        pred_mz, pred_intensity, target_mz, target_intensity, target_mask)
    return _tc_finish(dot_p, na_p, nb_p)[0, 0]
